# baseline (device time: 47707 ns/iter reference)
import jax
import jax.numpy as jnp
from jax import lax
from jax.experimental import pallas as pl
from jax.experimental.pallas import tpu as pltpu

SQ = 512
D = 1024
HQ = 8
HKV = 2
DH = 128
NCHUNK = 8
HPC = HQ // NCHUNK
SCALE = 0.08838834764831843

_CompilerParams = getattr(pltpu, "CompilerParams", None) or getattr(
    pltpu, "TPUCompilerParams"
)


def kernel(x, Wq, Wo, K_ext, V_ext):
    def body(
        x_ref,
        wq_ref,
        wo_ref,
        k_ref,
        v_ref,
        out_ref,
        o_bf,
        l_acc,
        recv_o,
        recv_l,
        o_send_sems,
        o_recv_sems,
        l_send_sems,
        l_recv_sems,
    ):
        my = lax.axis_index("i")
        partners = (my ^ 1, 3 - my)

        barrier = pltpu.get_barrier_semaphore()
        for nbr in partners:
            pl.semaphore_signal(
                barrier,
                inc=1,
                device_id=(nbr,),
                device_id_type=pl.DeviceIdType.MESH,
            )
        pl.semaphore_wait(barrier, 2)

        x_bf = x_ref[0].astype(jnp.bfloat16)

        def compute_chunk(t):
            qc = (
                jnp.dot(
                    x_bf,
                    wq_ref[:, t * HPC * DH : (t + 1) * HPC * DH].astype(
                        jnp.bfloat16
                    ),
                    preferred_element_type=jnp.float32,
                )
                * SCALE
            ).astype(jnp.bfloat16)
            kh = k_ref[0, :, t // 4, :].astype(jnp.bfloat16)
            vh = v_ref[0, :, t // 4, :]
            for hi in range(HPC):
                qh = qc[:, hi * DH : (hi + 1) * DH]
                s = lax.dot_general(
                    qh,
                    kh,
                    (((1,), (1,)), ((), ())),
                    preferred_element_type=jnp.float32,
                )
                p = jnp.exp(s)
                o = jnp.dot(p, vh, preferred_element_type=jnp.float32)
                o_bf[t, hi] = o.astype(jnp.bfloat16)
                l_acc[t, hi, :] = jnp.sum(p, axis=1)

        def make_exchange(stage, t):
            ro = pltpu.make_async_remote_copy(
                src_ref=o_bf.at[t],
                dst_ref=recv_o.at[stage, t],
                send_sem=o_send_sems.at[stage, t],
                recv_sem=o_recv_sems.at[stage, t],
                device_id=(partners[stage],),
                device_id_type=pl.DeviceIdType.MESH,
            )
            rl = pltpu.make_async_remote_copy(
                src_ref=l_acc.at[t],
                dst_ref=recv_l.at[stage, t],
                send_sem=l_send_sems.at[stage, t],
                recv_sem=l_recv_sems.at[stage, t],
                device_id=(partners[stage],),
                device_id_type=pl.DeviceIdType.MESH,
            )
            ro.start()
            rl.start()
            return ro, rl

        def wait_merge0(ex, t):
            ro, rl = ex
            ro.wait()
            rl.wait()
            l_acc[t] = l_acc[t] + recv_l[0, t]
            o_bf[t] = (
                o_bf[t].astype(jnp.float32)
                + recv_o[0, t].astype(jnp.float32)
            ).astype(jnp.bfloat16)

        def wait1(ex):
            ro, rl = ex
            ro.wait()
            rl.wait()

        def project_chunk(t, acc):
            for hi in range(HPC):
                h = t * HPC + hi
                of = o_bf[t, hi].astype(jnp.float32) + recv_o[
                    1, t, hi
                ].astype(jnp.float32)
                lh = l_acc[t, hi] + recv_l[1, t, hi]
                oh = of / lh[:, None]
                acc = acc + jnp.dot(
                    oh.astype(jnp.bfloat16),
                    wo_ref[h * DH : (h + 1) * DH, :].astype(jnp.bfloat16),
                    preferred_element_type=jnp.float32,
                )
            return acc

        ex0 = [None] * NCHUNK
        ex1 = [None] * NCHUNK

        compute_chunk(0)
        ex0[0] = make_exchange(0, 0)
        for t in range(1, NCHUNK):
            compute_chunk(t)
            ex0[t] = make_exchange(0, t)
            wait_merge0(ex0[t - 1], t - 1)
            ex1[t - 1] = make_exchange(1, t - 1)

        acc = jnp.zeros((SQ, D), jnp.float32)
        wait1(ex1[0])
        acc = project_chunk(0, acc)
        wait1(ex1[1])
        acc = project_chunk(1, acc)
        wait_merge0(ex0[NCHUNK - 1], NCHUNK - 1)
        ex1[NCHUNK - 1] = make_exchange(1, NCHUNK - 1)
        for t in range(2, NCHUNK):
            wait1(ex1[t])
            acc = project_chunk(t, acc)
        out_ref[0] = acc

    return pl.pallas_call(
        body,
        out_shape=jax.ShapeDtypeStruct((1, SQ, D), jnp.float32),
        in_specs=[pl.BlockSpec(memory_space=pltpu.VMEM)] * 5,
        out_specs=pl.BlockSpec(memory_space=pltpu.VMEM),
        scratch_shapes=[
            pltpu.VMEM((NCHUNK, HPC, SQ, DH), jnp.bfloat16),
            pltpu.VMEM((NCHUNK, HPC, SQ), jnp.float32),
            pltpu.VMEM((2, NCHUNK, HPC, SQ, DH), jnp.bfloat16),
            pltpu.VMEM((2, NCHUNK, HPC, SQ), jnp.float32),
            pltpu.SemaphoreType.DMA((2, NCHUNK)),
            pltpu.SemaphoreType.DMA((2, NCHUNK)),
            pltpu.SemaphoreType.DMA((2, NCHUNK)),
            pltpu.SemaphoreType.DMA((2, NCHUNK)),
        ],
        compiler_params=_CompilerParams(collective_id=0),
    )(x, Wq, Wo, K_ext, V_ext)


# device time: 44317 ns/iter; 1.0765x vs baseline; 1.0765x over previous
import jax
import jax.numpy as jnp
from jax import lax
from jax.experimental import pallas as pl
from jax.experimental.pallas import tpu as pltpu

SQ = 512
D = 1024
HQ = 8
HKV = 2
DH = 128
NCHUNK = 4
HPC = HQ // NCHUNK
SCALE = 0.08838834764831843

_CompilerParams = getattr(pltpu, "CompilerParams", None) or getattr(
    pltpu, "TPUCompilerParams"
)


def kernel(x, Wq, Wo, K_ext, V_ext):
    def body(
        x_ref,
        wq_ref,
        wo_ref,
        k_ref,
        v_ref,
        out_ref,
        o_acc,
        o_bf,
        l_acc,
        recv_o,
        recv_l,
        o_send_sems,
        o_recv_sems,
        l_send_sems,
        l_recv_sems,
    ):
        my = lax.axis_index("i")
        partners = (my ^ 1, 3 - my)

        barrier = pltpu.get_barrier_semaphore()
        for nbr in partners:
            pl.semaphore_signal(
                barrier,
                inc=1,
                device_id=(nbr,),
                device_id_type=pl.DeviceIdType.MESH,
            )
        pl.semaphore_wait(barrier, 2)

        x_bf = x_ref[0].astype(jnp.bfloat16)

        def compute_chunk(t):
            qc = (
                jnp.dot(
                    x_bf,
                    wq_ref[:, t * HPC * DH : (t + 1) * HPC * DH].astype(
                        jnp.bfloat16
                    ),
                    preferred_element_type=jnp.float32,
                )
                * SCALE
            ).astype(jnp.bfloat16)
            kh = k_ref[0, :, t // 2, :].astype(jnp.bfloat16)
            vh = v_ref[0, :, t // 2, :]
            for hi in range(HPC):
                qh = qc[:, hi * DH : (hi + 1) * DH]
                s = lax.dot_general(
                    qh,
                    kh,
                    (((1,), (1,)), ((), ())),
                    preferred_element_type=jnp.float32,
                )
                p = jnp.exp(s)
                o = jnp.dot(p, vh, preferred_element_type=jnp.float32)
                o_acc[t, hi] = o
                o_bf[t, hi] = o.astype(jnp.bfloat16)
                l_acc[t, hi, :] = jnp.sum(p, axis=1)

        def make_exchange(stage, t):
            ro = pltpu.make_async_remote_copy(
                src_ref=o_bf.at[t],
                dst_ref=recv_o.at[stage, t],
                send_sem=o_send_sems.at[stage, t],
                recv_sem=o_recv_sems.at[stage, t],
                device_id=(partners[stage],),
                device_id_type=pl.DeviceIdType.MESH,
            )
            rl = pltpu.make_async_remote_copy(
                src_ref=l_acc.at[t],
                dst_ref=recv_l.at[stage, t],
                send_sem=l_send_sems.at[stage, t],
                recv_sem=l_recv_sems.at[stage, t],
                device_id=(partners[stage],),
                device_id_type=pl.DeviceIdType.MESH,
            )
            ro.start()
            rl.start()
            return ro, rl

        def wait_merge(ex, stage, t):
            ro, rl = ex
            ro.wait()
            rl.wait()
            l_acc[t] = l_acc[t] + recv_l[stage, t]
            merged = o_acc[t] + recv_o[stage, t].astype(jnp.float32)
            o_acc[t] = merged
            if stage == 0:
                o_bf[t] = merged.astype(jnp.bfloat16)

        def project_chunk(t, acc):
            for hi in range(HPC):
                h = t * HPC + hi
                oh = o_acc[t, hi] / l_acc[t, hi][:, None]
                acc = acc + jnp.dot(
                    oh.astype(jnp.bfloat16),
                    wo_ref[h * DH : (h + 1) * DH, :].astype(jnp.bfloat16),
                    preferred_element_type=jnp.float32,
                )
            return acc

        ex0 = [None] * NCHUNK
        ex1 = [None] * NCHUNK

        compute_chunk(0)
        ex0[0] = make_exchange(0, 0)

        compute_chunk(1)
        ex0[1] = make_exchange(0, 1)
        wait_merge(ex0[0], 0, 0)
        ex1[0] = make_exchange(1, 0)

        compute_chunk(2)
        ex0[2] = make_exchange(0, 2)
        wait_merge(ex0[1], 0, 1)
        ex1[1] = make_exchange(1, 1)
        wait_merge(ex1[0], 1, 0)

        compute_chunk(3)
        ex0[3] = make_exchange(0, 3)
        wait_merge(ex0[2], 0, 2)
        ex1[2] = make_exchange(1, 2)
        wait_merge(ex1[1], 1, 1)

        acc = project_chunk(0, jnp.zeros((SQ, D), jnp.float32))
        acc = project_chunk(1, acc)
        wait_merge(ex0[3], 0, 3)
        ex1[3] = make_exchange(1, 3)
        wait_merge(ex1[2], 1, 2)
        acc = project_chunk(2, acc)
        wait_merge(ex1[3], 1, 3)
        out_ref[0] = project_chunk(3, acc)

    return pl.pallas_call(
        body,
        out_shape=jax.ShapeDtypeStruct((1, SQ, D), jnp.float32),
        in_specs=[pl.BlockSpec(memory_space=pltpu.VMEM)] * 5,
        out_specs=pl.BlockSpec(memory_space=pltpu.VMEM),
        scratch_shapes=[
            pltpu.VMEM((NCHUNK, HPC, SQ, DH), jnp.float32),
            pltpu.VMEM((NCHUNK, HPC, SQ, DH), jnp.bfloat16),
            pltpu.VMEM((NCHUNK, HPC, SQ), jnp.float32),
            pltpu.VMEM((2, NCHUNK, HPC, SQ, DH), jnp.bfloat16),
            pltpu.VMEM((2, NCHUNK, HPC, SQ), jnp.float32),
            pltpu.SemaphoreType.DMA((2, NCHUNK)),
            pltpu.SemaphoreType.DMA((2, NCHUNK)),
            pltpu.SemaphoreType.DMA((2, NCHUNK)),
            pltpu.SemaphoreType.DMA((2, NCHUNK)),
        ],
        compiler_params=_CompilerParams(collective_id=0),
    )(x, Wq, Wo, K_ext, V_ext)
